# baseline (speedup 1.0000x reference)
import functools

import jax
import jax.numpy as jnp
from jax import lax
from jax.experimental import pallas as pl
from jax.experimental.pallas import tpu as pltpu
from jax.experimental.pallas import tpu_sc as plsc

B, N, D = 4, 8192, 1024
BN = 512
NB = N // BN

SEGS = 8
SEG_BINS = N // SEGS
LANES = 16
IDX_CHUNKS = D // LANES
BIN_CHUNKS = SEG_BINS // LANES


def _maxarg_body(x_ref, vals_ref, inds_ref):
    nb = pl.program_id(1)
    x = x_ref[0]
    m = jnp.max(x, axis=0)
    iota = lax.broadcasted_iota(jnp.int32, (BN, D), 0)
    loc = jnp.min(jnp.where(x == m[None, :], iota, BN), axis=0) + nb * BN

    @pl.when(nb == 0)
    def _():
        vals_ref[0, 0] = m
        inds_ref[0, 0] = loc

    @pl.when(nb != 0)
    def _():
        cur = vals_ref[0, 0]
        take = m > cur
        vals_ref[0, 0] = jnp.where(take, m, cur)
        inds_ref[0, 0] = jnp.where(take, loc, inds_ref[0, 0])


def _maxarg(x, interpret=False):
    vals3, inds3 = pl.pallas_call(
        _maxarg_body,
        grid=(B, NB),
        in_specs=[pl.BlockSpec((1, BN, D), lambda b, nb: (b, nb, 0))],
        out_specs=[
            pl.BlockSpec((1, 1, D), lambda b, nb: (b, 0, 0)),
            pl.BlockSpec((1, 1, D), lambda b, nb: (b, 0, 0)),
        ],
        out_shape=[
            jax.ShapeDtypeStruct((B, 1, D), jnp.float32),
            jax.ShapeDtypeStruct((B, 1, D), jnp.int32),
        ],
        interpret=interpret,
    )(x)
    return vals3.reshape(B, D), inds3.reshape(B, D)


def _hist_body(inds_hbm, mask_hbm, scores_hbm, idx_v, mask_v, hist_v):
    c = lax.axis_index("c")
    s = lax.axis_index("s")
    b = c * 2 + s // 8
    seg = s % 8
    lo = seg * SEG_BINS

    pltpu.sync_copy(inds_hbm.at[b], idx_v)
    pltpu.sync_copy(mask_hbm.at[b], mask_v)

    def zero_body(j, carry):
        hist_v[pl.ds(j * LANES, LANES)] = jnp.zeros((LANES,), jnp.float32)
        return carry

    lax.fori_loop(0, BIN_CHUNKS, zero_body, 0)

    def scat_body(j, carry):
        idx = idx_v[pl.ds(j * LANES, LANES)]
        rel = idx - lo
        inr = (rel >= 0) & (rel < SEG_BINS)
        relc = jnp.clip(rel, 0, SEG_BINS - 1)
        counts, last = plsc.scan_count(relc, mask=inr)
        plsc.addupdate_scatter(hist_v, [relc], counts.astype(jnp.float32),
                               mask=last & inr)
        return carry

    lax.fori_loop(0, IDX_CHUNKS, scat_body, 0)

    def tot_body(j, tacc):
        idx = idx_v[pl.ds(j * LANES, LANES)]
        return tacc + plsc.load_gather(mask_v, [idx])

    tot = lax.fori_loop(0, IDX_CHUNKS, tot_body,
                        jnp.zeros((LANES,), jnp.int32))
    recip_v = jnp.full((LANES,), 1.0, jnp.float32) / jnp.full(
        (LANES,), jnp.sum(tot).astype(jnp.float32), jnp.float32)

    def norm_body(j, carry):
        sl = pl.ds(j * LANES, LANES)
        mk = mask_v[pl.ds(lo + j * LANES, LANES)]
        hist_v[sl] = jnp.where(mk == 0, 0.0, hist_v[sl]) * recip_v
        return carry

    lax.fori_loop(0, BIN_CHUNKS, norm_body, 0)

    pltpu.sync_copy(hist_v, scores_hbm.at[b, pl.ds(lo, SEG_BINS)])


@functools.cache
def _hist():
    return pl.kernel(
        _hist_body,
        mesh=plsc.VectorSubcoreMesh(core_axis_name="c", subcore_axis_name="s"),
        out_type=jax.ShapeDtypeStruct((B, N), jnp.float32),
        compiler_params=pltpu.CompilerParams(needs_layout_passes=False),
        scratch_types=[
            pltpu.VMEM((D,), jnp.int32),
            pltpu.VMEM((N,), jnp.int32),
            pltpu.VMEM((SEG_BINS,), jnp.float32),
        ],
    )


@jax.jit
def kernel(token_embeddings, attention_mask):
    pooled_vals, pooled_inds = _maxarg(token_embeddings)
    scores = _hist()(pooled_inds, attention_mask)
    return scores, pooled_vals


# speedup vs baseline: 1.4119x; 1.4119x over previous
import functools

import jax
import jax.numpy as jnp
from jax import lax
from jax.experimental import pallas as pl
from jax.experimental.pallas import tpu as pltpu
from jax.experimental.pallas import tpu_sc as plsc

B, N, D = 4, 8192, 1024
BN = 1024
NB = N // BN

SEGS = 8
SEG_BINS = N // SEGS
LANES = 16
IDX_CHUNKS = D // LANES
BIN_CHUNKS = SEG_BINS // LANES


def _maxarg_body(x_ref, vals_ref, inds_ref):
    nb = pl.program_id(0)
    x = x_ref[...]
    m = jnp.max(x, axis=1)
    iota = lax.broadcasted_iota(jnp.int32, (B, BN, D), 1)
    loc = jnp.min(jnp.where(x == m[:, None, :], iota, BN), axis=1) + nb * BN

    @pl.when(nb == 0)
    def _():
        vals_ref[...] = m
        inds_ref[...] = loc

    @pl.when(nb != 0)
    def _():
        cur = vals_ref[...]
        take = m > cur
        vals_ref[...] = jnp.where(take, m, cur)
        inds_ref[...] = jnp.where(take, loc, inds_ref[...])


def _maxarg(x, interpret=False):
    return pl.pallas_call(
        _maxarg_body,
        grid=(NB,),
        in_specs=[pl.BlockSpec((B, BN, D), lambda nb: (0, nb, 0))],
        out_specs=[
            pl.BlockSpec((B, D), lambda nb: (0, 0)),
            pl.BlockSpec((B, D), lambda nb: (0, 0)),
        ],
        out_shape=[
            jax.ShapeDtypeStruct((B, D), jnp.float32),
            jax.ShapeDtypeStruct((B, D), jnp.int32),
        ],
        interpret=interpret,
    )(x)


def _hist_body(inds_hbm, mask_hbm, scores_hbm, idx_v, mask_v, hist_v):
    c = lax.axis_index("c")
    s = lax.axis_index("s")
    b = c * 2 + s // 8
    seg = s % 8
    lo = seg * SEG_BINS

    pltpu.sync_copy(inds_hbm.at[b], idx_v)
    pltpu.sync_copy(mask_hbm.at[b], mask_v)

    def zero_body(j, carry):
        hist_v[pl.ds(j * LANES, LANES)] = jnp.zeros((LANES,), jnp.float32)
        return carry

    lax.fori_loop(0, BIN_CHUNKS, zero_body, 0)

    def scat_body(j, carry):
        idx = idx_v[pl.ds(j * LANES, LANES)]
        rel = idx - lo
        inr = (rel >= 0) & (rel < SEG_BINS)
        relc = jnp.clip(rel, 0, SEG_BINS - 1)
        counts, last = plsc.scan_count(relc, mask=inr)
        plsc.addupdate_scatter(hist_v, [relc], counts.astype(jnp.float32),
                               mask=last & inr)
        return carry

    lax.fori_loop(0, IDX_CHUNKS, scat_body, 0)

    def tot_body(j, tacc):
        idx = idx_v[pl.ds(j * LANES, LANES)]
        return tacc + plsc.load_gather(mask_v, [idx])

    tot = lax.fori_loop(0, IDX_CHUNKS, tot_body,
                        jnp.zeros((LANES,), jnp.int32))
    recip_v = jnp.full((LANES,), 1.0, jnp.float32) / jnp.full(
        (LANES,), jnp.sum(tot).astype(jnp.float32), jnp.float32)

    def norm_body(j, carry):
        sl = pl.ds(j * LANES, LANES)
        mk = mask_v[pl.ds(lo + j * LANES, LANES)]
        hist_v[sl] = jnp.where(mk == 0, 0.0, hist_v[sl]) * recip_v
        return carry

    lax.fori_loop(0, BIN_CHUNKS, norm_body, 0)

    pltpu.sync_copy(hist_v, scores_hbm.at[b, pl.ds(lo, SEG_BINS)])


@functools.cache
def _hist():
    return pl.kernel(
        _hist_body,
        mesh=plsc.VectorSubcoreMesh(core_axis_name="c", subcore_axis_name="s"),
        out_type=jax.ShapeDtypeStruct((B, N), jnp.float32),
        compiler_params=pltpu.CompilerParams(needs_layout_passes=False),
        scratch_types=[
            pltpu.VMEM((D,), jnp.int32),
            pltpu.VMEM((N,), jnp.int32),
            pltpu.VMEM((SEG_BINS,), jnp.float32),
        ],
    )


@jax.jit
def kernel(token_embeddings, attention_mask):
    pooled_vals, pooled_inds = _maxarg(token_embeddings)
    scores = _hist()(pooled_inds, attention_mask)
    return scores, pooled_vals


# speedup vs baseline: 1.4182x; 1.0045x over previous
import functools

import jax
import jax.numpy as jnp
from jax import lax
from jax.experimental import pallas as pl
from jax.experimental.pallas import tpu as pltpu
from jax.experimental.pallas import tpu_sc as plsc

B, N, D = 4, 8192, 1024
BN = 1024
NB = N // BN

SEGS = 8
SEG_BINS = N // SEGS
LANES = 16
IDX_CHUNKS = D // LANES
BIN_CHUNKS = SEG_BINS // LANES


def _maxarg_body(x_ref, vals_ref, inds_ref):
    nb = pl.program_id(0)
    x = x_ref[...]
    m = jnp.max(x, axis=1)
    iota = lax.broadcasted_iota(jnp.int32, (B, BN, D), 1).astype(jnp.float32)
    locf = jnp.min(jnp.where(x == m[:, None, :], iota, float(BN)), axis=1)
    loc = locf.astype(jnp.int32) + nb * BN

    @pl.when(nb == 0)
    def _():
        vals_ref[...] = m
        inds_ref[...] = loc

    @pl.when(nb != 0)
    def _():
        cur = vals_ref[...]
        take = m > cur
        vals_ref[...] = jnp.where(take, m, cur)
        inds_ref[...] = jnp.where(take, loc, inds_ref[...])


def _maxarg(x, interpret=False):
    return pl.pallas_call(
        _maxarg_body,
        grid=(NB,),
        in_specs=[pl.BlockSpec((B, BN, D), lambda nb: (0, nb, 0))],
        out_specs=[
            pl.BlockSpec((B, D), lambda nb: (0, 0)),
            pl.BlockSpec((B, D), lambda nb: (0, 0)),
        ],
        out_shape=[
            jax.ShapeDtypeStruct((B, D), jnp.float32),
            jax.ShapeDtypeStruct((B, D), jnp.int32),
        ],
        interpret=interpret,
    )(x)


def _hist_body(inds_hbm, mask_hbm, scores_hbm, idx_v, mask_v, hist_v):
    c = lax.axis_index("c")
    s = lax.axis_index("s")
    b = c * 2 + s // 8
    seg = s % 8
    lo = seg * SEG_BINS

    pltpu.sync_copy(inds_hbm.at[b], idx_v)
    pltpu.sync_copy(mask_hbm.at[b], mask_v)

    def zero_body(j, carry):
        hist_v[pl.ds(j * LANES, LANES)] = jnp.zeros((LANES,), jnp.float32)
        return carry

    lax.fori_loop(0, BIN_CHUNKS, zero_body, 0)

    def scat_body(j, carry):
        idx = idx_v[pl.ds(j * LANES, LANES)]
        rel = idx - lo
        inr = (rel >= 0) & (rel < SEG_BINS)
        relc = jnp.clip(rel, 0, SEG_BINS - 1)
        counts, last = plsc.scan_count(relc, mask=inr)
        plsc.addupdate_scatter(hist_v, [relc], counts.astype(jnp.float32),
                               mask=last & inr)
        return carry

    lax.fori_loop(0, IDX_CHUNKS, scat_body, 0)

    def tot_body(j, tacc):
        idx = idx_v[pl.ds(j * LANES, LANES)]
        return tacc + plsc.load_gather(mask_v, [idx])

    tot = lax.fori_loop(0, IDX_CHUNKS, tot_body,
                        jnp.zeros((LANES,), jnp.int32))
    recip_v = jnp.full((LANES,), 1.0, jnp.float32) / jnp.full(
        (LANES,), jnp.sum(tot).astype(jnp.float32), jnp.float32)

    def norm_body(j, carry):
        sl = pl.ds(j * LANES, LANES)
        mk = mask_v[pl.ds(lo + j * LANES, LANES)]
        hist_v[sl] = jnp.where(mk == 0, 0.0, hist_v[sl]) * recip_v
        return carry

    lax.fori_loop(0, BIN_CHUNKS, norm_body, 0)

    pltpu.sync_copy(hist_v, scores_hbm.at[b, pl.ds(lo, SEG_BINS)])


@functools.cache
def _hist():
    return pl.kernel(
        _hist_body,
        mesh=plsc.VectorSubcoreMesh(core_axis_name="c", subcore_axis_name="s"),
        out_type=jax.ShapeDtypeStruct((B, N), jnp.float32),
        compiler_params=pltpu.CompilerParams(needs_layout_passes=False),
        scratch_types=[
            pltpu.VMEM((D,), jnp.int32),
            pltpu.VMEM((N,), jnp.int32),
            pltpu.VMEM((SEG_BINS,), jnp.float32),
        ],
    )


@jax.jit
def kernel(token_embeddings, attention_mask):
    pooled_vals, pooled_inds = _maxarg(token_embeddings)
    scores = _hist()(pooled_inds, attention_mask)
    return scores, pooled_vals
